# Initial kernel scaffold; baseline (speedup 1.0000x reference)
import jax
import jax.numpy as jnp
from jax.experimental import pallas as pl


def kernel(x, W1, b1, W2, b2, W3, b3, W4, b4, W5, b5, W6, b6, W7, b7, W8, b8, W9, b9, W10, b10, edge_index):
    raise NotImplementedError("write your pallas kernel here")


# speedup vs baseline: 6.9781x; 6.9781x over previous
import functools

import jax
import jax.numpy as jnp
from jax import lax
from jax.experimental import pallas as pl
from jax.experimental.pallas import tpu as pltpu
from jax.experimental.pallas import tpu_sc as plsc

_N = 10000
_NP = 10240
_E = 160000
_NC, _NS = 2, 16
_NW = _NC * _NS
_EW = 5120
_CH = 128
_NCH = _EW // _CH
_STR = _NP // _NS
_BM = 1024


def _mesh():
    return plsc.VectorSubcoreMesh(
        core_axis_name="c", subcore_axis_name="s",
        num_cores=_NC, num_subcores=_NS)


def _sc_agg(srcp, dstp, table):
    f = table.shape[1]

    @functools.partial(
        pl.kernel,
        out_type=jax.ShapeDtypeStruct((_NC, _NP, f), jnp.float32),
        mesh=_mesh(),
        scratch_types=[
            pltpu.VMEM((_NCH, _CH), jnp.int32),
            pltpu.VMEM((_NCH, _CH), jnp.int32),
            pltpu.VMEM((_CH, f), jnp.float32),
            pltpu.VMEM((64, f), jnp.float32),
            pltpu.VMEM_SHARED((_NP, f), jnp.float32),
            pltpu.SemaphoreType.DMA,
        ],
        compiler_params=pltpu.CompilerParams(use_tc_tiling_on_sc=False),
    )
    def k(srcp_hbm, dstp_hbm, table_hbm, out_hbm,
          src_v, dst_v, rows_v, zbuf, acc, sem):
        c = lax.axis_index("c")
        s = lax.axis_index("s")
        wid = c * _NS + s
        pltpu.sync_copy(srcp_hbm.at[wid], src_v)
        pltpu.sync_copy(dstp_hbm.at[wid], dst_v)

        zero = jnp.zeros((16,), jnp.float32)
        for r in range(64):
            for q in range(f // 16):
                zbuf[r, pl.ds(q * 16, 16)] = zero

        def zcp(t, carry):
            pltpu.sync_copy(zbuf, acc.at[pl.ds(s * _STR + t * 64, 64)])
            return carry
        lax.fori_loop(0, _STR // 64, zcp, 0)
        plsc.subcore_barrier()

        def chunk(j, carry):
            pltpu.async_copy(table_hbm.at[src_v.at[j]], rows_v, sem).wait()
            pltpu.sync_copy(rows_v, acc.at[dst_v.at[j]], add=True)
            return carry
        lax.fori_loop(0, _NCH, chunk, 0)
        plsc.subcore_barrier()

        pltpu.sync_copy(acc.at[pl.ds(s * _STR, _STR)],
                        out_hbm.at[c, pl.ds(s * _STR, _STR)])

    return k(srcp, dstp, table)


def _row_spec(f):
    return pl.BlockSpec((_BM, f), lambda i: (i, 0))


def _full_spec(a, b):
    return pl.BlockSpec((a, b), lambda i: (0, 0))


def _dinv(p0, p1):
    def body(p0_ref, p1_ref, o_ref):
        i = pl.program_id(0)
        rows = i * _BM + lax.broadcasted_iota(jnp.int32, (_BM, 16), 0)
        deg = p0_ref[...] + p1_ref[...] + 1.0
        o_ref[...] = jnp.where(rows < _N, lax.rsqrt(deg), 0.0)
    return pl.pallas_call(
        body, grid=(_NP // _BM,),
        in_specs=[_row_spec(16), _row_spec(16)],
        out_specs=_row_spec(16),
        out_shape=jax.ShapeDtypeStruct((_NP, 16), jnp.float32),
    )(p0, p1)


def _scale(h, d):
    f = h.shape[1]
    def body(h_ref, d_ref, o_ref):
        o_ref[...] = h_ref[...] * d_ref[...]
    return pl.pallas_call(
        body, grid=(_NP // _BM,),
        in_specs=[_row_spec(f), _row_spec(1)],
        out_specs=_row_spec(f),
        out_shape=jax.ShapeDtypeStruct((_NP, f), jnp.float32),
    )(h, d)


def _matA(p0, p1, t, d, W, b, skip=None):
    K, Nn = W.shape
    has_skip = skip is not None

    def body(*refs):
        if has_skip:
            p0r, p1r, tr, dr, wr, br, sr, outr = refs
        else:
            p0r, p1r, tr, dr, wr, br, outr = refs
        u = (p0r[...] + p1r[...] + tr[...]) * dr[...]
        acc = jnp.dot(u, wr[...], preferred_element_type=jnp.float32) + br[...]
        if has_skip:
            acc = acc + sr[...]
        outr[...] = jnp.maximum(acc, 0.0)

    in_specs = [_row_spec(K)] * 3 + [_row_spec(1), _full_spec(K, Nn),
                                     _full_spec(1, Nn)]
    args = [p0, p1, t, d, W, b]
    if has_skip:
        in_specs.append(_row_spec(Nn))
        args.append(skip)
    return pl.pallas_call(
        body, grid=(_NP // _BM,),
        in_specs=in_specs,
        out_specs=_row_spec(Nn),
        out_shape=jax.ShapeDtypeStruct((_NP, Nn), jnp.float32),
    )(*args)


def _matB1(h, d, W):
    K, Nn = W.shape
    def body(h_ref, d_ref, w_ref, o_ref):
        o_ref[...] = jnp.dot(h_ref[...], w_ref[...],
                             preferred_element_type=jnp.float32) * d_ref[...]
    return pl.pallas_call(
        body, grid=(_NP // _BM,),
        in_specs=[_row_spec(K), _row_spec(1), _full_spec(K, Nn)],
        out_specs=_row_spec(Nn),
        out_shape=jax.ShapeDtypeStruct((_NP, Nn), jnp.float32),
    )(h, d, W)


def _ewB2(p0, p1, t, d, b):
    f = t.shape[1]
    def body(p0r, p1r, tr, dr, br, outr):
        outr[...] = jnp.maximum(
            (p0r[...] + p1r[...] + tr[...]) * dr[...] + br[...], 0.0)
    return pl.pallas_call(
        body, grid=(_NP // _BM,),
        in_specs=[_row_spec(f)] * 3 + [_row_spec(1), _full_spec(1, f)],
        out_specs=_row_spec(f),
        out_shape=jax.ShapeDtypeStruct((_NP, f), jnp.float32),
    )(p0, p1, t, d, b)


def kernel(x, W1, b1, W2, b2, W3, b3, W4, b4, W5, b5, W6, b6, W7, b7,
           W8, b8, W9, b9, W10, b10, edge_index):
    src = edge_index[0]
    dst = edge_index[1]
    padi = jnp.full((_NW * _EW - _E,), _N, jnp.int32)
    srcp = jnp.concatenate([src, padi]).reshape(_NW, _NCH, _CH)
    dstp = jnp.concatenate([dst, padi]).reshape(_NW, _NCH, _CH)

    ones_col = (jnp.arange(_NP) < _N).astype(jnp.float32)[:, None]
    ones_tab = ones_col * jnp.ones((1, 16), jnp.float32)

    pd = _sc_agg(srcp, dstp, ones_tab)
    d = _dinv(pd[0], pd[1])[:, 0:1]

    xp = jnp.concatenate(
        [x, jnp.zeros((_NP - _N, x.shape[1]), jnp.float32)], axis=0)

    def layer_a(h, W, b, skip=None):
        t = _scale(h, d)
        p = _sc_agg(srcp, dstp, t)
        return _matA(p[0], p[1], t, d, W, b.reshape(1, -1), skip)

    def layer_b(h, W, b):
        t = _matB1(h, d, W)
        p = _sc_agg(srcp, dstp, t)
        return _ewB2(p[0], p[1], t, d, b.reshape(1, -1))

    x1 = layer_a(xp, W1, b1)
    x2 = layer_b(x1, W2, b2)
    x3 = layer_b(x2, W3, b3)
    x4 = layer_b(x3, W4, b4)
    x5 = layer_b(x4, W5, b5)
    x6 = layer_a(x5, W6, b6, x4)
    x7 = layer_a(x6, W7, b7, x3)
    x8 = layer_a(x7, W8, b8, x2)
    x9 = layer_a(x8, W9, b9, x1)
    x10 = layer_b(x9, W10, b10)
    return x10[:_N]


# speedup vs baseline: 7.5437x; 1.0811x over previous
import functools

import jax
import jax.numpy as jnp
from jax import lax
from jax.experimental import pallas as pl
from jax.experimental.pallas import tpu as pltpu
from jax.experimental.pallas import tpu_sc as plsc

_N = 10000
_NP = 10240
_E = 160000
_NC, _NS = 2, 16
_NW = _NC * _NS
_EW = 5120
_CH = 128
_NCH = _EW // _CH
_STR = _NP // _NS
_BM = 1024


def _mesh():
    return plsc.VectorSubcoreMesh(
        core_axis_name="c", subcore_axis_name="s",
        num_cores=_NC, num_subcores=_NS)


def _sc_agg(srcp, dstp, table):
    f = table.shape[1]
    nbuf = 2 if f == 128 else 8
    ngrp = _NCH // nbuf

    @functools.partial(
        pl.kernel,
        out_type=jax.ShapeDtypeStruct((_NC, _NP, f), jnp.float32),
        mesh=_mesh(),
        scratch_types=[
            pltpu.VMEM((_NCH, _CH), jnp.int32),
            pltpu.VMEM((_NCH, _CH), jnp.int32),
            [pltpu.VMEM((_CH, f), jnp.float32) for _ in range(nbuf)],
            pltpu.VMEM((32, f), jnp.float32),
            pltpu.VMEM_SHARED((_NP, f), jnp.float32),
            [pltpu.SemaphoreType.DMA for _ in range(nbuf)],
            [pltpu.SemaphoreType.DMA for _ in range(nbuf)],
        ],
        compiler_params=pltpu.CompilerParams(use_tc_tiling_on_sc=False),
    )
    def k(srcp_hbm, dstp_hbm, table_hbm, out_hbm,
          src_v, dst_v, rows, zbuf, acc, gsem, ssem):
        c = lax.axis_index("c")
        s = lax.axis_index("s")
        wid = c * _NS + s
        pltpu.sync_copy(srcp_hbm.at[wid], src_v)
        pltpu.sync_copy(dstp_hbm.at[wid], dst_v)

        zero = jnp.zeros((16,), jnp.float32)
        for r in range(32):
            for q in range(f // 16):
                zbuf[r, pl.ds(q * 16, 16)] = zero

        def zcp(t, carry):
            pltpu.sync_copy(zbuf, acc.at[pl.ds(s * _STR + t * 32, 32)])
            return carry
        lax.fori_loop(0, _STR // 32, zcp, 0)
        plsc.subcore_barrier()

        def group(g, carry):
            j0 = g * nbuf
            gd = [pltpu.async_copy(table_hbm.at[src_v.at[j0 + b]],
                                   rows[b], gsem[b]) for b in range(nbuf)]
            sd = []
            for b in range(nbuf):
                gd[b].wait()
                sd.append(pltpu.async_copy(rows[b], acc.at[dst_v.at[j0 + b]],
                                           ssem[b], add=True))
            for b in range(nbuf):
                sd[b].wait()
            return carry
        lax.fori_loop(0, ngrp, group, 0)
        plsc.subcore_barrier()

        pltpu.sync_copy(acc.at[pl.ds(s * _STR, _STR)],
                        out_hbm.at[c, pl.ds(s * _STR, _STR)])

    return k(srcp, dstp, table)


def _row_spec(f):
    return pl.BlockSpec((_BM, f), lambda i: (i, 0))


def _full_spec(a, b):
    return pl.BlockSpec((a, b), lambda i: (0, 0))


def _dinv(p0, p1):
    def body(p0_ref, p1_ref, o_ref):
        i = pl.program_id(0)
        rows = i * _BM + lax.broadcasted_iota(jnp.int32, (_BM, 16), 0)
        deg = p0_ref[...] + p1_ref[...] + 1.0
        o_ref[...] = jnp.where(rows < _N, lax.rsqrt(deg), 0.0)
    return pl.pallas_call(
        body, grid=(_NP // _BM,),
        in_specs=[_row_spec(16), _row_spec(16)],
        out_specs=_row_spec(16),
        out_shape=jax.ShapeDtypeStruct((_NP, 16), jnp.float32),
    )(p0, p1)


def _scale(h, d):
    f = h.shape[1]
    def body(h_ref, d_ref, o_ref):
        o_ref[...] = h_ref[...] * d_ref[...]
    return pl.pallas_call(
        body, grid=(_NP // _BM,),
        in_specs=[_row_spec(f), _row_spec(1)],
        out_specs=_row_spec(f),
        out_shape=jax.ShapeDtypeStruct((_NP, f), jnp.float32),
    )(h, d)


def _matA(p0, p1, t, d, W, b, skip=None):
    K, Nn = W.shape
    has_skip = skip is not None

    def body(*refs):
        if has_skip:
            p0r, p1r, tr, dr, wr, br, sr, outr = refs
        else:
            p0r, p1r, tr, dr, wr, br, outr = refs
        u = (p0r[...] + p1r[...] + tr[...]) * dr[...]
        acc = jnp.dot(u, wr[...], preferred_element_type=jnp.float32) + br[...]
        if has_skip:
            acc = acc + sr[...]
        outr[...] = jnp.maximum(acc, 0.0)

    in_specs = [_row_spec(K)] * 3 + [_row_spec(1), _full_spec(K, Nn),
                                     _full_spec(1, Nn)]
    args = [p0, p1, t, d, W, b]
    if has_skip:
        in_specs.append(_row_spec(Nn))
        args.append(skip)
    return pl.pallas_call(
        body, grid=(_NP // _BM,),
        in_specs=in_specs,
        out_specs=_row_spec(Nn),
        out_shape=jax.ShapeDtypeStruct((_NP, Nn), jnp.float32),
    )(*args)


def _matB1(h, d, W):
    K, Nn = W.shape
    def body(h_ref, d_ref, w_ref, o_ref):
        o_ref[...] = jnp.dot(h_ref[...], w_ref[...],
                             preferred_element_type=jnp.float32) * d_ref[...]
    return pl.pallas_call(
        body, grid=(_NP // _BM,),
        in_specs=[_row_spec(K), _row_spec(1), _full_spec(K, Nn)],
        out_specs=_row_spec(Nn),
        out_shape=jax.ShapeDtypeStruct((_NP, Nn), jnp.float32),
    )(h, d, W)


def _ewB2(p0, p1, t, d, b):
    f = t.shape[1]
    def body(p0r, p1r, tr, dr, br, outr):
        outr[...] = jnp.maximum(
            (p0r[...] + p1r[...] + tr[...]) * dr[...] + br[...], 0.0)
    return pl.pallas_call(
        body, grid=(_NP // _BM,),
        in_specs=[_row_spec(f)] * 3 + [_row_spec(1), _full_spec(1, f)],
        out_specs=_row_spec(f),
        out_shape=jax.ShapeDtypeStruct((_NP, f), jnp.float32),
    )(p0, p1, t, d, b)


def kernel(x, W1, b1, W2, b2, W3, b3, W4, b4, W5, b5, W6, b6, W7, b7,
           W8, b8, W9, b9, W10, b10, edge_index):
    src = edge_index[0]
    dst = edge_index[1]
    padi = jnp.full((_NW * _EW - _E,), _N, jnp.int32)
    srcp = jnp.concatenate([src, padi]).reshape(_NW, _NCH, _CH)
    dstp = jnp.concatenate([dst, padi]).reshape(_NW, _NCH, _CH)

    ones_col = (jnp.arange(_NP) < _N).astype(jnp.float32)[:, None]
    ones_tab = ones_col * jnp.ones((1, 16), jnp.float32)

    pd = _sc_agg(srcp, dstp, ones_tab)
    d = _dinv(pd[0], pd[1])[:, 0:1]

    xp = jnp.concatenate(
        [x, jnp.zeros((_NP - _N, x.shape[1]), jnp.float32)], axis=0)

    def layer_a(h, W, b, skip=None):
        t = _scale(h, d)
        p = _sc_agg(srcp, dstp, t)
        return _matA(p[0], p[1], t, d, W, b.reshape(1, -1), skip)

    def layer_b(h, W, b):
        t = _matB1(h, d, W)
        p = _sc_agg(srcp, dstp, t)
        return _ewB2(p[0], p[1], t, d, b.reshape(1, -1))

    x1 = layer_a(xp, W1, b1)
    x2 = layer_b(x1, W2, b2)
    x3 = layer_b(x2, W3, b3)
    x4 = layer_b(x3, W4, b4)
    x5 = layer_b(x4, W5, b5)
    x6 = layer_a(x5, W6, b6, x4)
    x7 = layer_a(x6, W7, b7, x3)
    x8 = layer_a(x7, W8, b8, x2)
    x9 = layer_a(x8, W9, b9, x1)
    x10 = layer_b(x9, W10, b10)
    return x10[:_N]


# speedup vs baseline: 16.5735x; 2.1970x over previous
import functools

import jax
import jax.numpy as jnp
from jax import lax
from jax.experimental import pallas as pl
from jax.experimental.pallas import tpu as pltpu
from jax.experimental.pallas import tpu_sc as plsc

_N = 10000
_NP = 10240
_E = 160000
_NC, _NS = 2, 16
_NW = _NC * _NS
_EW = 5120
_CH = 128
_NCH = _EW // _CH
_STR = _NP // _NS
_BM = 1024


def _mesh():
    return plsc.VectorSubcoreMesh(
        core_axis_name="c", subcore_axis_name="s",
        num_cores=_NC, num_subcores=_NS)


def _sc_agg(srcp, dstp, table):
    f = table.shape[1]
    nbuf = 2 if f == 128 else 8
    ngrp = _NCH // nbuf

    @functools.partial(
        pl.kernel,
        out_type=jax.ShapeDtypeStruct((_NC, _NP, f), jnp.float32),
        mesh=_mesh(),
        scratch_types=[
            pltpu.VMEM((_NCH, _CH), jnp.int32),
            pltpu.VMEM((_NCH, _CH), jnp.int32),
            [pltpu.VMEM((_CH, f), jnp.float32) for _ in range(nbuf)],
            pltpu.VMEM((32, f), jnp.float32),
            pltpu.VMEM_SHARED((_NP, f), jnp.float32),
            [pltpu.SemaphoreType.DMA for _ in range(nbuf)],
            [pltpu.SemaphoreType.DMA for _ in range(nbuf)],
        ],
        compiler_params=pltpu.CompilerParams(use_tc_tiling_on_sc=False),
    )
    def k(srcp_hbm, dstp_hbm, table_hbm, out_hbm,
          src_v, dst_v, rows, zbuf, acc, gsem, ssem):
        c = lax.axis_index("c")
        s = lax.axis_index("s")
        wid = c * _NS + s
        pltpu.sync_copy(srcp_hbm.at[wid], src_v)
        pltpu.sync_copy(dstp_hbm.at[wid], dst_v)

        zero = jnp.zeros((16,), jnp.float32)
        for r in range(32):
            for q in range(f // 16):
                zbuf[r, pl.ds(q * 16, 16)] = zero

        def zcp(t, carry):
            pltpu.sync_copy(zbuf, acc.at[pl.ds(s * _STR + t * 32, 32)])
            return carry
        lax.fori_loop(0, _STR // 32, zcp, 0)
        plsc.subcore_barrier()

        def group(g, carry):
            j0 = g * nbuf
            gd = [pltpu.async_copy(table_hbm.at[src_v.at[j0 + b]],
                                   rows[b], gsem[b]) for b in range(nbuf)]
            sd = []
            for b in range(nbuf):
                gd[b].wait()
                sd.append(pltpu.async_copy(rows[b], acc.at[dst_v.at[j0 + b]],
                                           ssem[b], add=True))
            for b in range(nbuf):
                sd[b].wait()
            return carry
        lax.fori_loop(0, ngrp, group, 0)
        plsc.subcore_barrier()

        pltpu.sync_copy(acc.at[pl.ds(s * _STR, _STR)],
                        out_hbm.at[c, pl.ds(s * _STR, _STR)])

    return k(srcp, dstp, table)


def _row_spec(f):
    return pl.BlockSpec((_BM, f), lambda i: (i, 0))


def _full_spec(a, b):
    return pl.BlockSpec((a, b), lambda i: (0, 0))


def _dinv(p0, p1):
    def body(p0_ref, p1_ref, o_ref):
        i = pl.program_id(0)
        rows = i * _BM + lax.broadcasted_iota(jnp.int32, (_BM, 16), 0)
        deg = p0_ref[...] + p1_ref[...] + 1.0
        o_ref[...] = jnp.where(rows < _N, lax.rsqrt(deg), 0.0)
    return pl.pallas_call(
        body, grid=(_NP // _BM,),
        in_specs=[_row_spec(16), _row_spec(16)],
        out_specs=_row_spec(16),
        out_shape=jax.ShapeDtypeStruct((_NP, 16), jnp.float32),
    )(p0, p1)


def _scale(h, d):
    f = h.shape[1]
    def body(h_ref, d_ref, o_ref):
        o_ref[...] = h_ref[...] * d_ref[...]
    return pl.pallas_call(
        body, grid=(_NP // _BM,),
        in_specs=[_row_spec(f), _row_spec(1)],
        out_specs=_row_spec(f),
        out_shape=jax.ShapeDtypeStruct((_NP, f), jnp.float32),
    )(h, d)


def _matA(p0, p1, t, d, W, b, skip=None):
    K, Nn = W.shape
    has_skip = skip is not None

    def body(*refs):
        if has_skip:
            p0r, p1r, tr, dr, wr, br, sr, outr = refs
        else:
            p0r, p1r, tr, dr, wr, br, outr = refs
        u = (p0r[...] + p1r[...] + tr[...]) * dr[...]
        acc = jnp.dot(u, wr[...], preferred_element_type=jnp.float32) + br[...]
        if has_skip:
            acc = acc + sr[...]
        outr[...] = jnp.maximum(acc, 0.0)

    in_specs = [_row_spec(K)] * 3 + [_row_spec(1), _full_spec(K, Nn),
                                     _full_spec(1, Nn)]
    args = [p0, p1, t, d, W, b]
    if has_skip:
        in_specs.append(_row_spec(Nn))
        args.append(skip)
    return pl.pallas_call(
        body, grid=(_NP // _BM,),
        in_specs=in_specs,
        out_specs=_row_spec(Nn),
        out_shape=jax.ShapeDtypeStruct((_NP, Nn), jnp.float32),
    )(*args)


def _matB1(h, d, W):
    K, Nn = W.shape
    def body(h_ref, d_ref, w_ref, o_ref):
        o_ref[...] = jnp.dot(h_ref[...], w_ref[...],
                             preferred_element_type=jnp.float32) * d_ref[...]
    return pl.pallas_call(
        body, grid=(_NP // _BM,),
        in_specs=[_row_spec(K), _row_spec(1), _full_spec(K, Nn)],
        out_specs=_row_spec(Nn),
        out_shape=jax.ShapeDtypeStruct((_NP, Nn), jnp.float32),
    )(h, d, W)


def _ewB2(p0, p1, t, d, b):
    f = t.shape[1]
    def body(p0r, p1r, tr, dr, br, outr):
        outr[...] = jnp.maximum(
            (p0r[...] + p1r[...] + tr[...]) * dr[...] + br[...], 0.0)
    return pl.pallas_call(
        body, grid=(_NP // _BM,),
        in_specs=[_row_spec(f)] * 3 + [_row_spec(1), _full_spec(1, f)],
        out_specs=_row_spec(f),
        out_shape=jax.ShapeDtypeStruct((_NP, f), jnp.float32),
    )(p0, p1, t, d, b)


def kernel(x, W1, b1, W2, b2, W3, b3, W4, b4, W5, b5, W6, b6, W7, b7,
           W8, b8, W9, b9, W10, b10, edge_index):
    src = edge_index[0]
    dst = edge_index[1]
    padi = _N + (jnp.arange(_NW * _EW - _E, dtype=jnp.int32) % (_NP - _N))
    srcp = jnp.concatenate([src, padi]).reshape(_NW, _NCH, _CH)
    dstp = jnp.concatenate([dst, padi]).reshape(_NW, _NCH, _CH)

    ones_col = (jnp.arange(_NP) < _N).astype(jnp.float32)[:, None]
    ones_tab = ones_col * jnp.ones((1, 16), jnp.float32)

    pd = _sc_agg(srcp, dstp, ones_tab)
    d = _dinv(pd[0], pd[1])[:, 0:1]

    xp = jnp.concatenate(
        [x, jnp.zeros((_NP - _N, x.shape[1]), jnp.float32)], axis=0)

    def layer_a(h, W, b, skip=None):
        t = _scale(h, d)
        p = _sc_agg(srcp, dstp, t)
        return _matA(p[0], p[1], t, d, W, b.reshape(1, -1), skip)

    def layer_b(h, W, b):
        t = _matB1(h, d, W)
        p = _sc_agg(srcp, dstp, t)
        return _ewB2(p[0], p[1], t, d, b.reshape(1, -1))

    x1 = layer_a(xp, W1, b1)
    x2 = layer_b(x1, W2, b2)
    x3 = layer_b(x2, W3, b3)
    x4 = layer_b(x3, W4, b4)
    x5 = layer_b(x4, W5, b5)
    x6 = layer_a(x5, W6, b6, x4)
    x7 = layer_a(x6, W7, b7, x3)
    x8 = layer_a(x7, W8, b8, x2)
    x9 = layer_a(x8, W9, b9, x1)
    x10 = layer_b(x9, W10, b10)
    return x10[:_N]


# speedup vs baseline: 20.3666x; 1.2289x over previous
import functools

import jax
import jax.numpy as jnp
from jax import lax
from jax.experimental import pallas as pl
from jax.experimental.pallas import tpu as pltpu
from jax.experimental.pallas import tpu_sc as plsc

_N = 10000
_NP = 10240
_E = 160000
_NC, _NS = 2, 16
_NW = _NC * _NS
_EW = 5120
_CH = 128
_NCH = _EW // _CH
_STR = _NP // _NS
_BM = 1024


def _mesh():
    return plsc.VectorSubcoreMesh(
        core_axis_name="c", subcore_axis_name="s",
        num_cores=_NC, num_subcores=_NS)


def _sc_agg(srcp, dstp, table):
    f = table.shape[1]
    nbuf = 2 if f == 128 else 8
    ngrp = _NCH // nbuf

    @functools.partial(
        pl.kernel,
        out_type=jax.ShapeDtypeStruct((_NC, _NP, f), jnp.float32),
        mesh=_mesh(),
        scratch_types=[
            pltpu.VMEM((_NCH, _CH), jnp.int32),
            pltpu.VMEM((_NCH, _CH), jnp.int32),
            [pltpu.VMEM((_CH, f), jnp.float32) for _ in range(nbuf)],
            pltpu.VMEM((32, f), jnp.float32),
            pltpu.VMEM_SHARED((_NP, f), jnp.float32),
            [pltpu.SemaphoreType.DMA for _ in range(nbuf)],
            [pltpu.SemaphoreType.DMA for _ in range(nbuf)],
        ],
        compiler_params=pltpu.CompilerParams(use_tc_tiling_on_sc=False),
    )
    def k(srcp_hbm, dstp_hbm, table_hbm, out_hbm,
          src_v, dst_v, rows, zbuf, acc, gsem, ssem):
        c = lax.axis_index("c")
        s = lax.axis_index("s")
        wid = c * _NS + s
        pltpu.sync_copy(srcp_hbm.at[wid], src_v)
        pltpu.sync_copy(dstp_hbm.at[wid], dst_v)

        zero = jnp.zeros((16,), jnp.float32)
        for r in range(32):
            for q in range(f // 16):
                zbuf[r, pl.ds(q * 16, 16)] = zero

        def zcp(t, carry):
            pltpu.sync_copy(zbuf, acc.at[pl.ds(s * _STR + t * 32, 32)])
            return carry
        lax.fori_loop(0, _STR // 32, zcp, 0)
        plsc.subcore_barrier()

        def group(g, carry):
            j0 = g * nbuf
            gd = [pltpu.async_copy(table_hbm.at[src_v.at[j0 + b]],
                                   rows[b], gsem[b]) for b in range(nbuf)]
            sd = []
            for b in range(nbuf):
                gd[b].wait()
                sd.append(pltpu.async_copy(rows[b], acc.at[dst_v.at[j0 + b]],
                                           ssem[b], add=True))
            for b in range(nbuf):
                sd[b].wait()
            return carry
        lax.fori_loop(0, ngrp, group, 0)
        plsc.subcore_barrier()

        pltpu.sync_copy(acc.at[pl.ds(s * _STR, _STR)],
                        out_hbm.at[c, pl.ds(s * _STR, _STR)])

    return k(srcp, dstp, table)


def _row_spec(f):
    return pl.BlockSpec((_BM, f), lambda i: (i, 0))


def _pair_spec(f):
    return pl.BlockSpec((2, _BM, f), lambda i: (0, i, 0))


def _full_spec(a, b):
    return pl.BlockSpec((a, b), lambda i: (0, 0))


def _dinv_scale(pdeg, xpad):
    fx = xpad.shape[1]
    def body(pr, xr, dref, tref):
        i = pl.program_id(0)
        rows = i * _BM + lax.broadcasted_iota(jnp.int32, (_BM, 16), 0)
        dd = jnp.where(rows < _N, lax.rsqrt(pr[0] + pr[1] + 1.0), 0.0)
        dref[...] = dd
        tref[...] = xr[...] * dd[:, 0:1]
    return pl.pallas_call(
        body, grid=(_NP // _BM,),
        in_specs=[_pair_spec(16), _row_spec(fx)],
        out_specs=[_row_spec(16), _row_spec(fx)],
        out_shape=[jax.ShapeDtypeStruct((_NP, 16), jnp.float32),
                   jax.ShapeDtypeStruct((_NP, fx), jnp.float32)],
    )(pdeg, xpad)


def _fused(p, t, d, b, prev_W=None, skip=None, next_W=None,
           next_scale=False, want_x=False):
    K = t.shape[1]
    nout = prev_W.shape[1] if prev_W is not None else K
    want_t = next_W is not None or next_scale
    tn = next_W.shape[1] if next_W is not None else nout

    def body(*refs):
        it = iter(refs)
        pr = next(it); tr = next(it); dr = next(it); br = next(it)
        wr = next(it) if prev_W is not None else None
        sr = next(it) if skip is not None else None
        wnr = next(it) if next_W is not None else None
        xr = next(it) if want_x else None
        tnr = next(it) if want_t else None
        u = (pr[0] + pr[1] + tr[...]) * dr[...]
        if wr is not None:
            x = jnp.dot(u, wr[...], preferred_element_type=jnp.float32) \
                + br[...]
        else:
            x = u + br[...]
        if sr is not None:
            x = x + sr[...]
        x = jnp.maximum(x, 0.0)
        if xr is not None:
            xr[...] = x
        if tnr is not None:
            if wnr is not None:
                tnr[...] = jnp.dot(x, wnr[...],
                                   preferred_element_type=jnp.float32) \
                    * dr[...]
            else:
                tnr[...] = x * dr[...]

    in_specs = [_pair_spec(K), _row_spec(K), _row_spec(1), _full_spec(1, nout)]
    args = [p, t, d, b.reshape(1, -1)]
    if prev_W is not None:
        in_specs.append(_full_spec(K, nout)); args.append(prev_W)
    if skip is not None:
        in_specs.append(_row_spec(nout)); args.append(skip)
    if next_W is not None:
        in_specs.append(_full_spec(nout, tn)); args.append(next_W)
    out_specs, out_shape = [], []
    if want_x:
        out_specs.append(_row_spec(nout))
        out_shape.append(jax.ShapeDtypeStruct((_NP, nout), jnp.float32))
    if want_t:
        out_specs.append(_row_spec(tn))
        out_shape.append(jax.ShapeDtypeStruct((_NP, tn), jnp.float32))
    res = pl.pallas_call(
        body, grid=(_NP // _BM,),
        in_specs=in_specs, out_specs=out_specs, out_shape=out_shape,
    )(*args)
    return res if isinstance(res, (tuple, list)) else (res,)


def kernel(x, W1, b1, W2, b2, W3, b3, W4, b4, W5, b5, W6, b6, W7, b7,
           W8, b8, W9, b9, W10, b10, edge_index):
    src = edge_index[0]
    dst = edge_index[1]
    padi = _N + (jnp.arange(_NW * _EW - _E, dtype=jnp.int32) % (_NP - _N))
    srcp = jnp.concatenate([src, padi]).reshape(_NW, _NCH, _CH)
    dstp = jnp.concatenate([dst, padi]).reshape(_NW, _NCH, _CH)

    ones_col = (jnp.arange(_NP) < _N).astype(jnp.float32)[:, None]
    ones_tab = ones_col * jnp.ones((1, 16), jnp.float32)

    pdeg = _sc_agg(srcp, dstp, ones_tab)
    xp = jnp.concatenate(
        [x, jnp.zeros((_NP - _N, x.shape[1]), jnp.float32)], axis=0)
    d16, t1 = _dinv_scale(pdeg, xp)
    d = d16[:, 0:1]

    def agg(t):
        return _sc_agg(srcp, dstp, t)

    p = agg(t1)
    x1, t2 = _fused(p, t1, d, b1, prev_W=W1, next_W=W2, want_x=True)
    p = agg(t2)
    x2, t3 = _fused(p, t2, d, b2, next_W=W3, want_x=True)
    p = agg(t3)
    x3, t4 = _fused(p, t3, d, b3, next_W=W4, want_x=True)
    p = agg(t4)
    x4, t5 = _fused(p, t4, d, b4, next_W=W5, want_x=True)
    p = agg(t5)
    (t6,) = _fused(p, t5, d, b5, next_scale=True)
    p = agg(t6)
    (t7,) = _fused(p, t6, d, b6, prev_W=W6, skip=x4, next_scale=True)
    p = agg(t7)
    (t8,) = _fused(p, t7, d, b7, prev_W=W7, skip=x3, next_scale=True)
    p = agg(t8)
    (t9,) = _fused(p, t8, d, b8, prev_W=W8, skip=x2, next_scale=True)
    p = agg(t9)
    (t10,) = _fused(p, t9, d, b9, prev_W=W9, skip=x1, next_W=W10)
    p = agg(t10)
    (x10,) = _fused(p, t10, d, b10, want_x=True)
    return x10[:_N]


# speedup vs baseline: 22.7531x; 1.1172x over previous
import functools

import jax
import jax.numpy as jnp
from jax import lax
from jax.experimental import pallas as pl
from jax.experimental.pallas import tpu as pltpu
from jax.experimental.pallas import tpu_sc as plsc

_N = 10000
_NP = 10240
_E = 160000
_NC, _NS = 2, 16
_NW = _NC * _NS
_EW = 5120
_CH = 128
_NCH = _EW // _CH
_STR = _NP // _NS
_BM = 1024


def _mesh():
    return plsc.VectorSubcoreMesh(
        core_axis_name="c", subcore_axis_name="s",
        num_cores=_NC, num_subcores=_NS)


def _sc_agg(srcp, dstp, table):
    f = table.shape[1]
    dt = table.dtype
    nbuf = 2 if (f == 128 and dt == jnp.float32) else 8
    ngrp = _NCH // nbuf
    vl = 16 if dt == jnp.float32 else 32

    @functools.partial(
        pl.kernel,
        out_type=jax.ShapeDtypeStruct((_NC, _NP, f), dt),
        mesh=_mesh(),
        scratch_types=[
            pltpu.VMEM((_NCH, _CH), jnp.int32),
            pltpu.VMEM((_NCH, _CH), jnp.int32),
            [pltpu.VMEM((_CH, f), dt) for _ in range(nbuf)],
            pltpu.VMEM((32, f), dt),
            pltpu.VMEM_SHARED((_NP, f), dt),
            [pltpu.SemaphoreType.DMA for _ in range(nbuf)],
            [pltpu.SemaphoreType.DMA for _ in range(nbuf)],
        ],
        compiler_params=pltpu.CompilerParams(use_tc_tiling_on_sc=False),
    )
    def k(srcp_hbm, dstp_hbm, table_hbm, out_hbm,
          src_v, dst_v, rows, zbuf, acc, gsem, ssem):
        c = lax.axis_index("c")
        s = lax.axis_index("s")
        wid = c * _NS + s
        pltpu.sync_copy(srcp_hbm.at[wid], src_v)
        pltpu.sync_copy(dstp_hbm.at[wid], dst_v)

        zero = jnp.zeros((vl,), dt)
        for r in range(32):
            for q in range(f // vl):
                zbuf[r, pl.ds(q * vl, vl)] = zero

        def zcp(t, carry):
            pltpu.sync_copy(zbuf, acc.at[pl.ds(s * _STR + t * 32, 32)])
            return carry
        lax.fori_loop(0, _STR // 32, zcp, 0)
        plsc.subcore_barrier()

        def group(g, carry):
            j0 = g * nbuf
            gd = [pltpu.async_copy(table_hbm.at[src_v.at[j0 + b]],
                                   rows[b], gsem[b]) for b in range(nbuf)]
            sd = []
            for b in range(nbuf):
                gd[b].wait()
                sd.append(pltpu.async_copy(rows[b], acc.at[dst_v.at[j0 + b]],
                                           ssem[b], add=True))
            for b in range(nbuf):
                sd[b].wait()
            return carry
        lax.fori_loop(0, ngrp, group, 0)
        plsc.subcore_barrier()

        pltpu.sync_copy(acc.at[pl.ds(s * _STR, _STR)],
                        out_hbm.at[c, pl.ds(s * _STR, _STR)])

    return k(srcp, dstp, table)


def _row_spec(f):
    return pl.BlockSpec((_BM, f), lambda i: (i, 0))


def _pair_spec(f):
    return pl.BlockSpec((2, _BM, f), lambda i: (0, i, 0))


def _full_spec(a, b):
    return pl.BlockSpec((a, b), lambda i: (0, 0))


def _dinv_scale(pdeg, xpad):
    fx = xpad.shape[1]
    def body(pr, xr, dref, tref):
        i = pl.program_id(0)
        rows = i * _BM + lax.broadcasted_iota(jnp.int32, (_BM, 16), 0)
        dd = jnp.where(rows < _N, lax.rsqrt(pr[0] + pr[1] + 1.0), 0.0)
        dref[...] = dd
        tref[...] = (xr[...] * dd[:, 0:1]).astype(tref.dtype)
    return pl.pallas_call(
        body, grid=(_NP // _BM,),
        in_specs=[_pair_spec(16), _row_spec(fx)],
        out_specs=[_row_spec(16), _row_spec(fx)],
        out_shape=[jax.ShapeDtypeStruct((_NP, 16), jnp.float32),
                   jax.ShapeDtypeStruct((_NP, fx), jnp.bfloat16)],
    )(pdeg, xpad)


def _fused(p, t, d, b, prev_W=None, skip=None, next_W=None,
           next_scale=False, want_x=False, t_dtype=jnp.bfloat16):
    K = t.shape[1]
    nout = prev_W.shape[1] if prev_W is not None else K
    want_t = next_W is not None or next_scale
    tn = next_W.shape[1] if next_W is not None else nout

    def body(*refs):
        it = iter(refs)
        pr = next(it); tr = next(it); dr = next(it); br = next(it)
        wr = next(it) if prev_W is not None else None
        sr = next(it) if skip is not None else None
        wnr = next(it) if next_W is not None else None
        xr = next(it) if want_x else None
        tnr = next(it) if want_t else None
        agg32 = (pr[0].astype(jnp.float32) + pr[1].astype(jnp.float32)
                 + tr[...].astype(jnp.float32))
        u = agg32 * dr[...]
        if wr is not None:
            x = jnp.dot(u, wr[...], preferred_element_type=jnp.float32) \
                + br[...]
        else:
            x = u + br[...]
        if sr is not None:
            x = x + sr[...]
        x = jnp.maximum(x, 0.0)
        if xr is not None:
            xr[...] = x
        if tnr is not None:
            if wnr is not None:
                tnr[...] = (jnp.dot(x, wnr[...],
                                    preferred_element_type=jnp.float32)
                            * dr[...]).astype(tnr.dtype)
            else:
                tnr[...] = (x * dr[...]).astype(tnr.dtype)

    in_specs = [_pair_spec(K), _row_spec(K), _row_spec(1), _full_spec(1, nout)]
    args = [p, t, d, b.reshape(1, -1)]
    if prev_W is not None:
        in_specs.append(_full_spec(K, nout)); args.append(prev_W)
    if skip is not None:
        in_specs.append(_row_spec(nout)); args.append(skip)
    if next_W is not None:
        in_specs.append(_full_spec(nout, tn)); args.append(next_W)
    out_specs, out_shape = [], []
    if want_x:
        out_specs.append(_row_spec(nout))
        out_shape.append(jax.ShapeDtypeStruct((_NP, nout), jnp.float32))
    if want_t:
        out_specs.append(_row_spec(tn))
        out_shape.append(jax.ShapeDtypeStruct((_NP, tn), t_dtype))
    res = pl.pallas_call(
        body, grid=(_NP // _BM,),
        in_specs=in_specs, out_specs=out_specs, out_shape=out_shape,
    )(*args)
    return res if isinstance(res, (tuple, list)) else (res,)


def kernel(x, W1, b1, W2, b2, W3, b3, W4, b4, W5, b5, W6, b6, W7, b7,
           W8, b8, W9, b9, W10, b10, edge_index):
    src = edge_index[0]
    dst = edge_index[1]
    padi = _N + (jnp.arange(_NW * _EW - _E, dtype=jnp.int32) % (_NP - _N))
    srcp = jnp.concatenate([src, padi]).reshape(_NW, _NCH, _CH)
    dstp = jnp.concatenate([dst, padi]).reshape(_NW, _NCH, _CH)

    ones_col = (jnp.arange(_NP) < _N).astype(jnp.float32)[:, None]
    ones_tab = ones_col * jnp.ones((1, 16), jnp.float32)

    pdeg = _sc_agg(srcp, dstp, ones_tab)
    xp = jnp.concatenate(
        [x, jnp.zeros((_NP - _N, x.shape[1]), jnp.float32)], axis=0)
    d16, t1 = _dinv_scale(pdeg, xp)
    d = d16[:, 0:1]

    def agg(t):
        return _sc_agg(srcp, dstp, t)

    p = agg(t1)
    x1, t2 = _fused(p, t1, d, b1, prev_W=W1, next_W=W2, want_x=True)
    p = agg(t2)
    x2, t3 = _fused(p, t2, d, b2, next_W=W3, want_x=True)
    p = agg(t3)
    x3, t4 = _fused(p, t3, d, b3, next_W=W4, want_x=True)
    p = agg(t4)
    x4, t5 = _fused(p, t4, d, b4, next_W=W5, want_x=True,
                    t_dtype=jnp.float32)
    p = agg(t5)
    (t6,) = _fused(p, t5, d, b5, next_scale=True, t_dtype=jnp.float32)
    p = agg(t6)
    (t7,) = _fused(p, t6, d, b6, prev_W=W6, skip=x4, next_scale=True)
    p = agg(t7)
    (t8,) = _fused(p, t7, d, b7, prev_W=W7, skip=x3, next_scale=True)
    p = agg(t8)
    (t9,) = _fused(p, t8, d, b8, prev_W=W8, skip=x2, next_scale=True)
    p = agg(t9)
    (t10,) = _fused(p, t9, d, b9, prev_W=W9, skip=x1, next_W=W10)
    p = agg(t10)
    (x10,) = _fused(p, t10, d, b10, want_x=True)
    return x10[:_N]


# speedup vs baseline: 22.9099x; 1.0069x over previous
import functools

import jax
import jax.numpy as jnp
from jax import lax
from jax.experimental import pallas as pl
from jax.experimental.pallas import tpu as pltpu
from jax.experimental.pallas import tpu_sc as plsc

_N = 10000
_NP = 10240
_E = 160000
_NC, _NS = 2, 16
_NW = _NC * _NS
_EW = 5120
_CH = 128
_NCH = _EW // _CH
_STR = _NP // _NS
_BM = 1024


def _mesh():
    return plsc.VectorSubcoreMesh(
        core_axis_name="c", subcore_axis_name="s",
        num_cores=_NC, num_subcores=_NS)


def _sc_agg(srcp, dstp, table):
    f = table.shape[1]
    dt = table.dtype
    nbuf = 2 if (f == 128 and dt == jnp.float32) else 8
    ngrp = _NCH // nbuf
    vl = 16 if dt == jnp.float32 else 32
    _ZR = 32 if (f == 128 and dt == jnp.float32) else 128

    @functools.partial(
        pl.kernel,
        out_type=jax.ShapeDtypeStruct((_NC, _NP, f), dt),
        mesh=_mesh(),
        scratch_types=[
            pltpu.VMEM((_NCH, _CH), jnp.int32),
            pltpu.VMEM((_NCH, _CH), jnp.int32),
            [pltpu.VMEM((_CH, f), dt) for _ in range(nbuf)],
            pltpu.VMEM((_ZR, f), dt),
            pltpu.VMEM_SHARED((_NP, f), dt),
            [pltpu.SemaphoreType.DMA for _ in range(nbuf)],
            [pltpu.SemaphoreType.DMA for _ in range(nbuf)],
        ],
        compiler_params=pltpu.CompilerParams(use_tc_tiling_on_sc=False),
    )
    def k(srcp_hbm, dstp_hbm, table_hbm, out_hbm,
          src_v, dst_v, rows, zbuf, acc, gsem, ssem):
        c = lax.axis_index("c")
        s = lax.axis_index("s")
        wid = c * _NS + s
        pltpu.sync_copy(srcp_hbm.at[wid], src_v)
        pltpu.sync_copy(dstp_hbm.at[wid], dst_v)

        zero = jnp.zeros((vl,), dt)
        for r in range(_ZR):
            for q in range(f // vl):
                zbuf[r, pl.ds(q * vl, vl)] = zero

        def zcp(t, carry):
            pltpu.sync_copy(zbuf, acc.at[pl.ds(s * _STR + t * _ZR, _ZR)])
            return carry
        lax.fori_loop(0, _STR // _ZR, zcp, 0)
        plsc.subcore_barrier()

        def group(g, carry):
            j0 = g * nbuf
            gd = [pltpu.async_copy(table_hbm.at[src_v.at[j0 + b]],
                                   rows[b], gsem[b]) for b in range(nbuf)]
            sd = []
            for b in range(nbuf):
                gd[b].wait()
                sd.append(pltpu.async_copy(rows[b], acc.at[dst_v.at[j0 + b]],
                                           ssem[b], add=True))
            for b in range(nbuf):
                sd[b].wait()
            return carry
        lax.fori_loop(0, ngrp, group, 0)
        plsc.subcore_barrier()

        pltpu.sync_copy(acc.at[pl.ds(s * _STR, _STR)],
                        out_hbm.at[c, pl.ds(s * _STR, _STR)])

    return k(srcp, dstp, table)


def _row_spec(f):
    return pl.BlockSpec((_BM, f), lambda i: (i, 0))


def _pair_spec(f):
    return pl.BlockSpec((2, _BM, f), lambda i: (0, i, 0))


def _full_spec(a, b):
    return pl.BlockSpec((a, b), lambda i: (0, 0))


def _dinv_scale(pdeg, xpad):
    fx = xpad.shape[1]
    def body(pr, xr, dref, tref):
        i = pl.program_id(0)
        rows = i * _BM + lax.broadcasted_iota(jnp.int32, (_BM, 16), 0)
        dd = jnp.where(rows < _N, lax.rsqrt(pr[0] + pr[1] + 1.0), 0.0)
        dref[...] = dd
        tref[...] = (xr[...] * dd[:, 0:1]).astype(tref.dtype)
    return pl.pallas_call(
        body, grid=(_NP // _BM,),
        in_specs=[_pair_spec(16), _row_spec(fx)],
        out_specs=[_row_spec(16), _row_spec(fx)],
        out_shape=[jax.ShapeDtypeStruct((_NP, 16), jnp.float32),
                   jax.ShapeDtypeStruct((_NP, fx), jnp.bfloat16)],
    )(pdeg, xpad)


def _fused(p, t, d, b, prev_W=None, skip=None, next_W=None,
           next_scale=False, want_x=False, t_dtype=jnp.bfloat16):
    K = t.shape[1]
    nout = prev_W.shape[1] if prev_W is not None else K
    want_t = next_W is not None or next_scale
    tn = next_W.shape[1] if next_W is not None else nout

    def body(*refs):
        it = iter(refs)
        pr = next(it); tr = next(it); dr = next(it); br = next(it)
        wr = next(it) if prev_W is not None else None
        sr = next(it) if skip is not None else None
        wnr = next(it) if next_W is not None else None
        xr = next(it) if want_x else None
        tnr = next(it) if want_t else None
        agg32 = (pr[0].astype(jnp.float32) + pr[1].astype(jnp.float32)
                 + tr[...].astype(jnp.float32))
        u = agg32 * dr[...]
        if wr is not None:
            x = jnp.dot(u, wr[...], preferred_element_type=jnp.float32) \
                + br[...]
        else:
            x = u + br[...]
        if sr is not None:
            x = x + sr[...]
        x = jnp.maximum(x, 0.0)
        if xr is not None:
            xr[...] = x
        if tnr is not None:
            if wnr is not None:
                tnr[...] = (jnp.dot(x, wnr[...],
                                    preferred_element_type=jnp.float32)
                            * dr[...]).astype(tnr.dtype)
            else:
                tnr[...] = (x * dr[...]).astype(tnr.dtype)

    in_specs = [_pair_spec(K), _row_spec(K), _row_spec(1), _full_spec(1, nout)]
    args = [p, t, d, b.reshape(1, -1)]
    if prev_W is not None:
        in_specs.append(_full_spec(K, nout)); args.append(prev_W)
    if skip is not None:
        in_specs.append(_row_spec(nout)); args.append(skip)
    if next_W is not None:
        in_specs.append(_full_spec(nout, tn)); args.append(next_W)
    out_specs, out_shape = [], []
    if want_x:
        out_specs.append(_row_spec(nout))
        out_shape.append(jax.ShapeDtypeStruct((_NP, nout), jnp.float32))
    if want_t:
        out_specs.append(_row_spec(tn))
        out_shape.append(jax.ShapeDtypeStruct((_NP, tn), t_dtype))
    res = pl.pallas_call(
        body, grid=(_NP // _BM,),
        in_specs=in_specs, out_specs=out_specs, out_shape=out_shape,
    )(*args)
    return res if isinstance(res, (tuple, list)) else (res,)


def kernel(x, W1, b1, W2, b2, W3, b3, W4, b4, W5, b5, W6, b6, W7, b7,
           W8, b8, W9, b9, W10, b10, edge_index):
    src = edge_index[0]
    dst = edge_index[1]
    padi = _N + (jnp.arange(_NW * _EW - _E, dtype=jnp.int32) % (_NP - _N))
    srcp = jnp.concatenate([src, padi]).reshape(_NW, _NCH, _CH)
    dstp = jnp.concatenate([dst, padi]).reshape(_NW, _NCH, _CH)

    ones_col = (jnp.arange(_NP) < _N).astype(jnp.float32)[:, None]
    ones_tab = ones_col * jnp.ones((1, 16), jnp.float32)

    pdeg = _sc_agg(srcp, dstp, ones_tab)
    xp = jnp.concatenate(
        [x, jnp.zeros((_NP - _N, x.shape[1]), jnp.float32)], axis=0)
    d16, t1 = _dinv_scale(pdeg, xp)
    d = d16[:, 0:1]

    def agg(t):
        return _sc_agg(srcp, dstp, t)

    p = agg(t1)
    x1, t2 = _fused(p, t1, d, b1, prev_W=W1, next_W=W2, want_x=True)
    p = agg(t2)
    x2, t3 = _fused(p, t2, d, b2, next_W=W3, want_x=True)
    p = agg(t3)
    x3, t4 = _fused(p, t3, d, b3, next_W=W4, want_x=True)
    p = agg(t4)
    x4, t5 = _fused(p, t4, d, b4, next_W=W5, want_x=True,
                    t_dtype=jnp.float32)
    p = agg(t5)
    (t6,) = _fused(p, t5, d, b5, next_scale=True, t_dtype=jnp.float32)
    p = agg(t6)
    (t7,) = _fused(p, t6, d, b6, prev_W=W6, skip=x4, next_scale=True)
    p = agg(t7)
    (t8,) = _fused(p, t7, d, b7, prev_W=W7, skip=x3, next_scale=True)
    p = agg(t8)
    (t9,) = _fused(p, t8, d, b8, prev_W=W8, skip=x2, next_scale=True)
    p = agg(t9)
    (t10,) = _fused(p, t9, d, b9, prev_W=W9, skip=x1, next_W=W10)
    p = agg(t10)
    (x10,) = _fused(p, t10, d, b10, want_x=True)
    return x10[:_N]


# speedup vs baseline: 22.9829x; 1.0032x over previous
import functools

import numpy as np
import jax
import jax.numpy as jnp
from jax import lax
from jax.experimental import pallas as pl
from jax.experimental.pallas import tpu as pltpu
from jax.experimental.pallas import tpu_sc as plsc

_N = 10000
_NP = 10240
_E = 160000
_NC, _NS = 2, 16
_NW = _NC * _NS
_EW = 5120
_CH = 128
_NCH = _EW // _CH
_STR = _NP // _NS
_BM = 1024


def _mesh():
    return plsc.VectorSubcoreMesh(
        core_axis_name="c", subcore_axis_name="s",
        num_cores=_NC, num_subcores=_NS)


def _sc_agg(srcp, dstp, table):
    f = table.shape[1]
    dt = table.dtype
    nbuf = 2 if (f == 128 and dt == jnp.float32) else 8
    ngrp = _NCH // nbuf
    vl = 16 if dt == jnp.float32 else 32
    _ZR = 32 if (f == 128 and dt == jnp.float32) else 128

    @functools.partial(
        pl.kernel,
        out_type=jax.ShapeDtypeStruct((_NC, _NP, f), dt),
        mesh=_mesh(),
        scratch_types=[
            pltpu.VMEM((_NCH, _CH), jnp.int32),
            pltpu.VMEM((_NCH, _CH), jnp.int32),
            [pltpu.VMEM((_CH, f), dt) for _ in range(nbuf)],
            pltpu.VMEM((_ZR, f), dt),
            pltpu.VMEM_SHARED((_NP, f), dt),
            [pltpu.SemaphoreType.DMA for _ in range(nbuf)],
            [pltpu.SemaphoreType.DMA for _ in range(nbuf)],
        ],
        compiler_params=pltpu.CompilerParams(use_tc_tiling_on_sc=False),
    )
    def k(srcp_hbm, dstp_hbm, table_hbm, out_hbm,
          src_v, dst_v, rows, zbuf, acc, gsem, ssem):
        c = lax.axis_index("c")
        s = lax.axis_index("s")
        wid = c * _NS + s
        pltpu.sync_copy(srcp_hbm.at[wid], src_v)
        pltpu.sync_copy(dstp_hbm.at[wid], dst_v)

        zero = jnp.zeros((vl,), dt)
        for r in range(_ZR):
            for q in range(f // vl):
                zbuf[r, pl.ds(q * vl, vl)] = zero

        def zcp(t, carry):
            pltpu.sync_copy(zbuf, acc.at[pl.ds(s * _STR + t * _ZR, _ZR)])
            return carry
        lax.fori_loop(0, _STR // _ZR, zcp, 0)
        plsc.subcore_barrier()

        def group(g, carry):
            j0 = g * nbuf
            gd = [pltpu.async_copy(table_hbm.at[src_v.at[j0 + b]],
                                   rows[b], gsem[b]) for b in range(nbuf)]
            sd = []
            for b in range(nbuf):
                gd[b].wait()
                sd.append(pltpu.async_copy(rows[b], acc.at[dst_v.at[j0 + b]],
                                           ssem[b], add=True))
            for b in range(nbuf):
                sd[b].wait()
            return carry
        lax.fori_loop(0, ngrp, group, 0)
        plsc.subcore_barrier()

        pltpu.sync_copy(acc.at[pl.ds(s * _STR, _STR)],
                        out_hbm.at[c, pl.ds(s * _STR, _STR)])

    return k(srcp, dstp, table)


def _row_spec(f):
    return pl.BlockSpec((_BM, f), lambda i: (i, 0))


def _pair_spec(f):
    return pl.BlockSpec((2, _BM, f), lambda i: (0, i, 0))


def _full_spec(a, b):
    return pl.BlockSpec((a, b), lambda i: (0, 0))


def _dinv_scale(pdeg, xpad):
    fx = xpad.shape[1]
    def body(pr, xr, dref, tref):
        i = pl.program_id(0)
        rows = i * _BM + lax.broadcasted_iota(jnp.int32, (_BM, 16), 0)
        dd = jnp.where(rows < _N, lax.rsqrt(pr[0] + pr[1] + 1.0), 0.0)
        dref[...] = dd
        tref[...] = (xr[...] * dd[:, 0:1]).astype(tref.dtype)
    return pl.pallas_call(
        body, grid=(_NP // _BM,),
        in_specs=[_pair_spec(16), _row_spec(fx)],
        out_specs=[_row_spec(16), _row_spec(fx)],
        out_shape=[jax.ShapeDtypeStruct((_NP, 16), jnp.float32),
                   jax.ShapeDtypeStruct((_NP, fx), jnp.bfloat16)],
    )(pdeg, xpad)


def _fused(p, t, d, b, prev_W=None, skip=None, next_W=None,
           next_scale=False, want_x=False, t_dtype=jnp.bfloat16):
    K = t.shape[1]
    nout = prev_W.shape[1] if prev_W is not None else K
    want_t = next_W is not None or next_scale
    tn = next_W.shape[1] if next_W is not None else nout

    def body(*refs):
        it = iter(refs)
        pr = next(it); tr = next(it); dr = next(it); br = next(it)
        wr = next(it) if prev_W is not None else None
        sr = next(it) if skip is not None else None
        wnr = next(it) if next_W is not None else None
        xr = next(it) if want_x else None
        tnr = next(it) if want_t else None
        agg32 = (pr[0].astype(jnp.float32) + pr[1].astype(jnp.float32)
                 + tr[...].astype(jnp.float32))
        u = agg32 * dr[...]
        if wr is not None:
            x = jnp.dot(u, wr[...], preferred_element_type=jnp.float32) \
                + br[...]
        else:
            x = u + br[...]
        if sr is not None:
            x = x + sr[...]
        x = jnp.maximum(x, 0.0)
        if xr is not None:
            xr[...] = x
        if tnr is not None:
            if wnr is not None:
                tnr[...] = (jnp.dot(x, wnr[...],
                                    preferred_element_type=jnp.float32)
                            * dr[...]).astype(tnr.dtype)
            else:
                tnr[...] = (x * dr[...]).astype(tnr.dtype)

    in_specs = [_pair_spec(K), _row_spec(K), _row_spec(1), _full_spec(1, nout)]
    args = [p, t, d, b.reshape(1, -1)]
    if prev_W is not None:
        in_specs.append(_full_spec(K, nout)); args.append(prev_W)
    if skip is not None:
        in_specs.append(_row_spec(nout)); args.append(skip)
    if next_W is not None:
        in_specs.append(_full_spec(nout, tn)); args.append(next_W)
    out_specs, out_shape = [], []
    if want_x:
        out_specs.append(_row_spec(nout))
        out_shape.append(jax.ShapeDtypeStruct((_NP, nout), jnp.float32))
    if want_t:
        out_specs.append(_row_spec(tn))
        out_shape.append(jax.ShapeDtypeStruct((_NP, tn), t_dtype))
    res = pl.pallas_call(
        body, grid=(_NP // _BM,),
        in_specs=in_specs, out_specs=out_specs, out_shape=out_shape,
    )(*args)
    return res if isinstance(res, (tuple, list)) else (res,)


def kernel(x, W1, b1, W2, b2, W3, b3, W4, b4, W5, b5, W6, b6, W7, b7,
           W8, b8, W9, b9, W10, b10, edge_index):
    src = edge_index[0]
    dst = edge_index[1]
    padi = jnp.asarray(
        _N + (np.arange(_NW * _EW - _E, dtype=np.int32) % (_NP - _N)))
    srcp = jnp.concatenate([src, padi]).reshape(_NW, _NCH, _CH)
    dstp = jnp.concatenate([dst, padi]).reshape(_NW, _NCH, _CH)

    ones_tab = jnp.asarray(np.where(
        np.arange(_NP)[:, None] < _N, 1.0, 0.0).astype(np.float32)
        * np.ones((1, 16), np.float32))

    pdeg = _sc_agg(srcp, dstp, ones_tab)
    xp = jnp.concatenate(
        [x, jnp.zeros((_NP - _N, x.shape[1]), jnp.float32)], axis=0)
    d16, t1 = _dinv_scale(pdeg, xp)
    d = d16[:, 0:1]

    def agg(t):
        return _sc_agg(srcp, dstp, t)

    p = agg(t1)
    x1, t2 = _fused(p, t1, d, b1, prev_W=W1, next_W=W2, want_x=True)
    p = agg(t2)
    x2, t3 = _fused(p, t2, d, b2, next_W=W3, want_x=True)
    p = agg(t3)
    x3, t4 = _fused(p, t3, d, b3, next_W=W4, want_x=True)
    p = agg(t4)
    x4, t5 = _fused(p, t4, d, b4, next_W=W5, want_x=True,
                    t_dtype=jnp.float32)
    p = agg(t5)
    (t6,) = _fused(p, t5, d, b5, next_scale=True, t_dtype=jnp.float32)
    p = agg(t6)
    (t7,) = _fused(p, t6, d, b6, prev_W=W6, skip=x4, next_scale=True)
    p = agg(t7)
    (t8,) = _fused(p, t7, d, b7, prev_W=W7, skip=x3, next_scale=True)
    p = agg(t8)
    (t9,) = _fused(p, t8, d, b8, prev_W=W8, skip=x2, next_scale=True)
    p = agg(t9)
    (t10,) = _fused(p, t9, d, b9, prev_W=W9, skip=x1, next_W=W10)
    p = agg(t10)
    (x10,) = _fused(p, t10, d, b10, want_x=True)
    return x10[:_N]
